# baseline (speedup 1.0000x reference)
import jax
import jax.numpy as jnp
from jax.experimental import pallas as pl
from jax.experimental.pallas import tpu as pltpu

_B = 8


def _conv_body(xh_ref, w1a_ref, w1b_ref, b1c_ref, w2c_ref, b2c_ref, out_ref,
               xp_ref, lhsa_ref, lhsb_ref, y1w_ref, a1p_ref, lhs2_ref,
               y2w_ref):
    f32 = jnp.float32
    zrow = jnp.zeros((1, 384), f32)
    for b in range(_B):
        xp_ref[b, 0:1, :] = zrow
        xp_ref[b, 65:66, :] = zrow
        xp_ref[b, :, 0:3] = jnp.zeros((66, 3), f32)
        xp_ref[b, :, 195:384] = jnp.zeros((66, 189), f32)
        xp_ref[b, 1:65, 3:195] = xh_ref[b]

    for b in range(_B):
        for ky in range(3):
            lhsa_ref[pl.ds(b * 64, 64), pl.ds(256 * ky, 256)] = \
                xp_ref[b, ky:ky + 64, 0:256]
            lhsb_ref[pl.ds(b * 64, 64), pl.ds(256 * ky, 256)] = \
                xp_ref[b, ky:ky + 64, 128:384]

    w1a = w1a_ref[...]
    w1b = w1b_ref[...]
    b1h = b1c_ref[0:1, 0:128]
    for m in range(_B // 2):
        ya = jnp.dot(lhsa_ref[pl.ds(128 * m, 128), :], w1a,
                     preferred_element_type=f32)
        yb = jnp.dot(lhsb_ref[pl.ds(128 * m, 128), :], w1b,
                     preferred_element_type=f32)
        for ch in range(3):
            wm = jnp.maximum(ya[:, 256 * ch:256 * ch + 128],
                             ya[:, 256 * ch + 128:256 * ch + 256])
            y1w_ref[pl.ds(512 * m + 64 * ch, 64), :] = wm[0:64]
            y1w_ref[pl.ds(512 * m + 256 + 64 * ch, 64), :] = wm[64:128]
        wm = jnp.maximum(yb[:, 0:128], yb[:, 128:256])
        y1w_ref[pl.ds(512 * m + 192, 64), :] = wm[0:64]
        y1w_ref[pl.ds(512 * m + 448, 64), :] = wm[64:128]

    zr2 = jnp.zeros((1, 640), f32)
    zc2 = jnp.zeros((34, 16), f32)
    zc3 = jnp.zeros((34, 112), f32)
    for b in range(_B):
        a1p_ref[b, 0:1, :] = zr2
        a1p_ref[b, 33:34, :] = zr2
        a1p_ref[b, :, 0:16] = zc2
        a1p_ref[b, :, 528:640] = zc3
        for c in range(4):
            base = b * 256 + c * 64
            pe = y1w_ref[pl.ds(base, 32, 2), :]
            po = y1w_ref[pl.ds(base + 1, 32, 2), :]
            a1p_ref[b, 1:33, pl.ds(16 + 128 * c, 128)] = \
                jnp.maximum(jnp.maximum(pe, po) + b1h, 0.0)

    for b in range(_B):
        for ky in range(3):
            for c2 in range(4):
                lhs2_ref[pl.ds(b * 128 + c2 * 32, 32), pl.ds(256 * ky, 256)] = \
                    a1p_ref[b, ky:ky + 32, pl.ds(128 * c2, 256)]

    w2c = w2c_ref[...]
    b2h = b2c_ref[0:1, 0:128]
    for m in range(_B):
        y = jnp.dot(lhs2_ref[pl.ds(128 * m, 128), :], w2c,
                    preferred_element_type=f32)
        y2w_ref[pl.ds(128 * m, 128), :] = \
            jnp.maximum(y[:, 0:128], y[:, 128:256])

    for b in range(_B):
        pe = y2w_ref[pl.ds(b * 128, 64, 2), :]
        po = y2w_ref[pl.ds(b * 128 + 1, 64, 2), :]
        m2 = jnp.maximum(jnp.maximum(pe, po) + b2h, 0.0)
        for c2 in range(4):
            out_ref[b, :, pl.ds(128 * c2, 128)] = m2[16 * c2:16 * c2 + 16, :]


def _conv_stack(xh, w1a, w1b, b1c, w2c, b2c):
    n = xh.shape[0]
    f32 = jnp.float32
    return pl.pallas_call(
        _conv_body,
        out_shape=jax.ShapeDtypeStruct((n, 16, 512), f32),
        grid_spec=pltpu.PrefetchScalarGridSpec(
            num_scalar_prefetch=0,
            grid=(n // _B,),
            in_specs=[
                pl.BlockSpec((_B, 64, 192), lambda i: (i, 0, 0)),
                pl.BlockSpec((768, 768), lambda i: (0, 0)),
                pl.BlockSpec((768, 256), lambda i: (0, 0)),
                pl.BlockSpec((1, 256), lambda i: (0, 0)),
                pl.BlockSpec((768, 256), lambda i: (0, 0)),
                pl.BlockSpec((1, 256), lambda i: (0, 0)),
            ],
            out_specs=pl.BlockSpec((_B, 16, 512), lambda i: (i, 0, 0)),
            scratch_shapes=[
                pltpu.VMEM((_B, 66, 384), f32),
                pltpu.VMEM((64 * _B, 768), f32),
                pltpu.VMEM((64 * _B, 768), f32),
                pltpu.VMEM((256 * _B, 128), f32),
                pltpu.VMEM((_B, 34, 640), f32),
                pltpu.VMEM((128 * _B, 768), f32),
                pltpu.VMEM((128 * _B, 128), f32),
            ],
        ),
        compiler_params=pltpu.CompilerParams(
            dimension_semantics=("parallel",)),
    )(xh, w1a, w1b, b1c, w2c, b2c)


def _mlp_body(x_ref, w1_ref, b1_ref, w2_ref, b2_ref, o_ref):
    f32 = jnp.float32
    h = jnp.dot(x_ref[:, 0, :], w1_ref[pl.ds(0, 512), :],
                preferred_element_type=f32)
    for r in range(1, 16):
        h = h + jnp.dot(x_ref[:, r, :], w1_ref[pl.ds(512 * r, 512), :],
                        preferred_element_type=f32)
    h = jnp.maximum(h + b1_ref[...], 0.0)
    lg = jnp.dot(h, w2_ref[...], preferred_element_type=f32)
    lg = lg + b2_ref[...]
    s = 1.0 / (1.0 + jnp.exp(lg[:, 0:1] - lg[:, 1:2]))
    o_ref[...] = jnp.concatenate([1.0 - s, s], axis=1)


def _mlp(feat, w1m, b1, w2, b2):
    n = feat.shape[0]
    h1 = w1m.shape[1]
    c = w2.shape[1]
    nb = min(256, n)
    return pl.pallas_call(
        _mlp_body,
        out_shape=jax.ShapeDtypeStruct((n, c), jnp.float32),
        grid_spec=pltpu.PrefetchScalarGridSpec(
            num_scalar_prefetch=0,
            grid=(n // nb,),
            in_specs=[
                pl.BlockSpec((nb, 16, 512), lambda i: (i, 0, 0)),
                pl.BlockSpec((8192, h1), lambda i: (0, 0)),
                pl.BlockSpec((1, h1), lambda i: (0, 0)),
                pl.BlockSpec((h1, c), lambda i: (0, 0)),
                pl.BlockSpec((1, c), lambda i: (0, 0)),
            ],
            out_specs=pl.BlockSpec((nb, c), lambda i: (i, 0)),
        ),
        compiler_params=pltpu.CompilerParams(
            dimension_semantics=("parallel",),
            vmem_limit_bytes=64 * 1024 * 1024),
    )(feat, w1m, b1.reshape(1, h1), w2, b2.reshape(1, c))


def _band_w1(w1, lane0, chunks):
    k = jnp.arange(768)
    ll = lane0 + (k % 256)
    pn = ll - 3
    p = pn // 3 + 1
    ncol = 256 * len(chunks)
    n = jnp.arange(ncol)
    c = jnp.asarray(chunks)[n // 256]
    m = n % 256
    wp = 2 * ((m % 128) // 16) + m // 128
    out = jnp.zeros((768, ncol), jnp.float32)
    for kx in range(3):
        for ci in range(3):
            v = jnp.broadcast_to(w1[:, kx, ci, :][:, None, :], (3, 256, 16))
            v = v.reshape(768, 16)
            v = jnp.tile(v, (1, ncol // 16))
            hit = ((pn[:, None] % 3 == ci) & (pn[:, None] >= 0)
                   & (ll[:, None] < 201)
                   & (p[:, None] - 16 * c[None, :] - wp[None, :] == kx))
            out = out + jnp.where(hit, v, 0.0)
    return out


def _band_w2(w2):
    k = jnp.arange(768)
    r = k % 256
    wl = r // 16
    n = jnp.arange(256)
    wp = 2 * ((n % 128) // 32) + n // 128
    out = jnp.zeros((768, 256), jnp.float32)
    for kx in range(3):
        v = w2[:, kx]
        v = jnp.broadcast_to(v[:, None, :, :], (3, 16, 16, 32))
        v = v.reshape(768, 32)
        v = jnp.tile(v, (1, 8))
        hit = (wl[:, None] - wp[None, :] == kx) & (r[:, None] < 160)
        out = out + jnp.where(hit, v, 0.0)
    return out


def kernel(x_nchw, w_conv1, b_conv1, w_conv2, b_conv2,
           w_fc1, b_fc1, w_fc2, b_fc2):
    n = x_nchw.shape[0]
    xh = jnp.transpose(x_nchw, (0, 2, 3, 1)).reshape(n, 64, 192)
    w1a = _band_w1(w_conv1, 0, (0, 1, 2))
    w1b = _band_w1(w_conv1, 128, (3,))
    w2c = _band_w2(w_conv2)
    b1c = jnp.tile(b_conv1, 16).reshape(1, 256)
    b2c = jnp.tile(b_conv2, 8).reshape(1, 256)
    feat = _conv_stack(xh, w1a, w1b, b1c, w2c, b2c)
    return _mlp(feat, w_fc1, b_fc1, w_fc2, b_fc2)


# speedup vs baseline: 1.1651x; 1.1651x over previous
import jax
import jax.numpy as jnp
from jax.experimental import pallas as pl
from jax.experimental.pallas import tpu as pltpu

_B = 32


def _conv_body(xh_ref, w1c_ref, b1c_ref, w2c_ref, b2c_ref, out_ref,
               xp_ref, lhs1_ref, y1w_ref, a1p_ref, lhs2_ref, y2w_ref):
    f32 = jnp.float32
    zrow = jnp.zeros((1, 256), f32)
    for b in range(_B):
        xp_ref[b, 0:1, :] = zrow
        xp_ref[b, 65:66, :] = zrow
        xp_ref[b, :, 0:3] = jnp.zeros((66, 3), f32)
        xp_ref[b, :, 195:256] = jnp.zeros((66, 61), f32)
        xp_ref[b, 1:65, 3:195] = xh_ref[b]

    for b in range(_B):
        for ky in range(3):
            for c in range(4):
                lhs1_ref[pl.ds(b * 256 + c * 64, 64), pl.ds(54 * ky, 54)] = \
                    xp_ref[b, ky:ky + 64, pl.ds(48 * c, 54)]

    w1c = w1c_ref[...]
    b1h = b1c_ref[0:1, 0:128]
    for m in range(2 * _B):
        y = jnp.dot(lhs1_ref[pl.ds(128 * m, 128), :], w1c,
                    preferred_element_type=f32)
        y1w_ref[pl.ds(128 * m, 128), :] = \
            jnp.maximum(y[:, 0:128], y[:, 128:256])

    zr2 = jnp.zeros((1, 640), f32)
    zc2 = jnp.zeros((34, 16), f32)
    zc3 = jnp.zeros((34, 112), f32)
    for b in range(_B):
        a1p_ref[b, 0:1, :] = zr2
        a1p_ref[b, 33:34, :] = zr2
        a1p_ref[b, :, 0:16] = zc2
        a1p_ref[b, :, 528:640] = zc3
        for c in range(4):
            base = b * 256 + c * 64
            pe = y1w_ref[pl.ds(base, 32, 2), :]
            po = y1w_ref[pl.ds(base + 1, 32, 2), :]
            a1p_ref[b, 1:33, pl.ds(16 + 128 * c, 128)] = \
                jnp.maximum(jnp.maximum(pe, po) + b1h, 0.0)

    for b in range(_B):
        for ky in range(3):
            for c2 in range(4):
                lhs2_ref[pl.ds(b * 128 + c2 * 32, 32), pl.ds(256 * ky, 256)] = \
                    a1p_ref[b, ky:ky + 32, pl.ds(128 * c2, 256)]

    w2c = w2c_ref[...]
    b2h = b2c_ref[0:1, 0:128]
    for m in range(_B):
        y = jnp.dot(lhs2_ref[pl.ds(128 * m, 128), :], w2c,
                    preferred_element_type=f32)
        y2w_ref[pl.ds(128 * m, 128), :] = \
            jnp.maximum(y[:, 0:128], y[:, 128:256])

    for b in range(_B):
        pe = y2w_ref[pl.ds(b * 128, 64, 2), :]
        po = y2w_ref[pl.ds(b * 128 + 1, 64, 2), :]
        m2 = jnp.maximum(jnp.maximum(pe, po) + b2h, 0.0)
        for c2 in range(4):
            out_ref[b, :, pl.ds(128 * c2, 128)] = m2[16 * c2:16 * c2 + 16, :]


def _conv_stack(xh, w1c, b1c, w2c, b2c):
    n = xh.shape[0]
    f32 = jnp.float32
    return pl.pallas_call(
        _conv_body,
        out_shape=jax.ShapeDtypeStruct((n, 16, 512), f32),
        grid_spec=pltpu.PrefetchScalarGridSpec(
            num_scalar_prefetch=0,
            grid=(n // _B,),
            in_specs=[
                pl.BlockSpec((_B, 64, 192), lambda i: (i, 0, 0)),
                pl.BlockSpec((162, 256), lambda i: (0, 0)),
                pl.BlockSpec((1, 256), lambda i: (0, 0)),
                pl.BlockSpec((768, 256), lambda i: (0, 0)),
                pl.BlockSpec((1, 256), lambda i: (0, 0)),
            ],
            out_specs=pl.BlockSpec((_B, 16, 512), lambda i: (i, 0, 0)),
            scratch_shapes=[
                pltpu.VMEM((_B, 66, 256), f32),
                pltpu.VMEM((256 * _B, 162), f32),
                pltpu.VMEM((256 * _B, 128), f32),
                pltpu.VMEM((_B, 34, 640), f32),
                pltpu.VMEM((128 * _B, 768), f32),
                pltpu.VMEM((128 * _B, 128), f32),
            ],
        ),
        compiler_params=pltpu.CompilerParams(
            dimension_semantics=("parallel",)),
    )(xh, w1c, b1c, w2c, b2c)


def _mlp_body(x_ref, w1_ref, b1_ref, w2_ref, b2_ref, o_ref):
    f32 = jnp.float32
    h = jnp.dot(x_ref[:, 0, :], w1_ref[pl.ds(0, 512), :],
                preferred_element_type=f32)
    for r in range(1, 16):
        h = h + jnp.dot(x_ref[:, r, :], w1_ref[pl.ds(512 * r, 512), :],
                        preferred_element_type=f32)
    h = jnp.maximum(h + b1_ref[...], 0.0)
    lg = jnp.dot(h, w2_ref[...], preferred_element_type=f32)
    lg = lg + b2_ref[...]
    s = 1.0 / (1.0 + jnp.exp(lg[:, 0:1] - lg[:, 1:2]))
    o_ref[...] = jnp.concatenate([1.0 - s, s], axis=1)


def _mlp(feat, w1m, b1, w2, b2):
    n = feat.shape[0]
    h1 = w1m.shape[1]
    c = w2.shape[1]
    nb = min(256, n)
    return pl.pallas_call(
        _mlp_body,
        out_shape=jax.ShapeDtypeStruct((n, c), jnp.float32),
        grid_spec=pltpu.PrefetchScalarGridSpec(
            num_scalar_prefetch=0,
            grid=(n // nb,),
            in_specs=[
                pl.BlockSpec((nb, 16, 512), lambda i: (i, 0, 0)),
                pl.BlockSpec((8192, h1), lambda i: (0, 0)),
                pl.BlockSpec((1, h1), lambda i: (0, 0)),
                pl.BlockSpec((h1, c), lambda i: (0, 0)),
                pl.BlockSpec((1, c), lambda i: (0, 0)),
            ],
            out_specs=pl.BlockSpec((nb, c), lambda i: (i, 0)),
        ),
        compiler_params=pltpu.CompilerParams(
            dimension_semantics=("parallel",),
            vmem_limit_bytes=64 * 1024 * 1024),
    )(feat, w1m, b1.reshape(1, h1), w2, b2.reshape(1, c))


def _band_w1(w1):
    j = (jnp.arange(162) % 54) // 3
    n = jnp.arange(256)
    wp = 2 * ((n % 128) // 16) + n // 128
    out = jnp.zeros((162, 256), jnp.float32)
    for kx in range(3):
        v = w1[:, kx]
        v = jnp.broadcast_to(v[:, None, :, :], (3, 18, 3, 16))
        v = v.reshape(162, 16)
        v = jnp.tile(v, (1, 16))
        out = out + jnp.where(j[:, None] - wp[None, :] == kx, v, 0.0)
    return out


def _band_w2(w2):
    k = jnp.arange(768)
    r = k % 256
    wl = r // 16
    n = jnp.arange(256)
    wp = 2 * ((n % 128) // 32) + n // 128
    out = jnp.zeros((768, 256), jnp.float32)
    for kx in range(3):
        v = w2[:, kx]
        v = jnp.broadcast_to(v[:, None, :, :], (3, 16, 16, 32))
        v = v.reshape(768, 32)
        v = jnp.tile(v, (1, 8))
        hit = (wl[:, None] - wp[None, :] == kx) & (r[:, None] < 160)
        out = out + jnp.where(hit, v, 0.0)
    return out


def kernel(x_nchw, w_conv1, b_conv1, w_conv2, b_conv2,
           w_fc1, b_fc1, w_fc2, b_fc2):
    n = x_nchw.shape[0]
    xh = jnp.transpose(x_nchw, (0, 2, 3, 1)).reshape(n, 64, 192)
    w1c = _band_w1(w_conv1)
    w2c = _band_w2(w_conv2)
    b1c = jnp.tile(b_conv1, 16).reshape(1, 256)
    b2c = jnp.tile(b_conv2, 8).reshape(1, 256)
    feat = _conv_stack(xh, w1c, b1c, w2c, b2c)
    return _mlp(feat, w_fc1, b_fc1, w_fc2, b_fc2)
